# baseline (speedup 1.0000x reference)
import jax
import jax.numpy as jnp
from jax import lax
from jax.experimental import pallas as pl
from jax.experimental.pallas import tpu as pltpu
from jax.experimental.pallas import tpu_sc as plsc

VOCAB_SIZE = 1000000
DIM = 32
HASH_SIZE = int(VOCAB_SIZE * (1 - 0.95))

NC = 2
NS = 16
LANES = 16
NW = NC * NS

ROWS = 16384
COLS = 26
RB = 128
NRB = ROWS // RB
RB_PER_W = NRB // NW
XCHUNK = RB * COLS
NCG = 4
GSLOT = 8 * RB
NBLK = RB_PER_W * NCG
NPAIR = NBLK // 2


def _sc_kernel(x_hbm, w_hbm, h_hbm, o_hbm,
               xe, xg0, xg1, hg0, hg1, wg0, wg1, rows0, rows1, t0, t1,
               sem_h, sem_r0, sem_r1, sem_t0, sem_t1):
    wid = lax.axis_index("s") * NC + lax.axis_index("c")
    rb0 = wid * RB_PER_W
    iota = lax.iota(jnp.int32, LANES)
    iota26 = iota * 26
    div = jnp.full((LANES,), HASH_SIZE, jnp.int32)

    xg = (xg0, xg1)
    hg = (hg0, hg1)
    wg = (wg0, wg1)
    rows = (rows0, rows1)
    tb = (t0, t1)
    sem_r = (sem_r0, sem_r1)
    sem_t = (sem_t0, sem_t1)

    def stage_x(b):
        rb = jnp.minimum(rb0 + (b >> 2), NRB - 1)
        pltpu.sync_copy(x_hbm.at[pl.ds(rb * XCHUNK, XCHUNK)], xe)

    def extract(b, slot):
        cg = b & 3
        nv = jnp.minimum(COLS - cg * 8, 8)

        def valid(j, _):
            base = cg * 8 + j
            for r8 in range(8):
                idx = iota26 + (r8 * 16 * 26) + base
                v = plsc.load_gather(xe, [idx])
                xg[slot][pl.ds(j * RB + r8 * 16, LANES)] = v
            return 0

        def filler(j, _):
            sbase = (rb0 + (b >> 2)) * (NCG * GSLOT) + (b & 3) * GSLOT
            for r8 in range(8):
                xg[slot][pl.ds(j * RB + r8 * 16, LANES)] = (
                    iota + (sbase + j * RB + r8 * 16))
            return 0

        lax.fori_loop(0, nv, valid, 0)
        lax.fori_loop(nv, 8, filler, 0)

    def fire_hash(slot):
        def f(j, _):
            pltpu.async_copy(
                h_hbm.at[xg[slot].at[pl.ds(j * RB, RB)]],
                hg[slot].at[pl.ds(j * RB, RB)], sem_h)
            return 0
        lax.fori_loop(0, 8, f, 0)

    def wait_hash(slot):
        pltpu.make_async_copy(
            x_hbm.at[pl.ds(0, GSLOT)], hg[slot], sem_h).wait()

    def rem(slot):
        def f(i, _):
            v = hg[slot][pl.ds(i * LANES, LANES)]
            wg[slot][pl.ds(i * LANES, LANES)] = lax.rem(v, div)
            return 0
        lax.fori_loop(0, GSLOT // LANES, f, 0)

    def fire_rows(slot):
        def f(j, _):
            pltpu.async_copy(
                w_hbm.at[wg[slot].at[pl.ds(j * RB, RB)]],
                rows[slot].at[pl.ds(j * RB, RB)], sem_r[slot])
            return 0
        lax.fori_loop(0, 8, f, 0)

    def wait_rows(slot):
        pltpu.make_async_copy(
            w_hbm.at[pl.ds(0, GSLOT)], rows[slot], sem_r[slot]).wait()

    def t_dma(tslot, c, rb):
        return pltpu.make_async_copy(
            tb[tslot], o_hbm.at[c].at[:, rb], sem_t[tslot])

    def transpose_out(b, slot):
        cg = b & 3
        rb = rb0 + (b >> 2)
        nv = jnp.minimum(COLS - cg * 8, 8)
        for j in range(8):
            @pl.when(j < nv)
            def _():
                t_dma(j % 2, 0, 0).wait()

                def drow(d, _):
                    dgv = d >> 3
                    dlv = d & 7
                    dc = jnp.full((LANES,), d, jnp.int32)
                    for r8 in range(8):
                        ir = iota + (j * RB + r8 * 16)
                        v = plsc.load_gather(rows[slot], [ir, dc])
                        tb[j % 2][dgv, dlv, pl.ds(r8 * 16, LANES)] = v
                    return 0

                lax.fori_loop(0, DIM, drow, 0)
                t_dma(j % 2, cg * 8 + j, rb).start()

    t_dma(0, 0, rb0).start()
    t_dma(1, 1, rb0).start()
    stage_x(0)
    extract(0, 0)
    fire_hash(0)

    def half(k, b, slot):
        wait_hash(slot)

        @pl.when((b & 3) == 3)
        def _():
            stage_x(b + 1)
        extract(b + 1, 1 - slot)
        fire_hash(1 - slot)
        rem(slot)
        fire_rows(slot)

        @pl.when(b > 0)
        def _():
            wait_rows(1 - slot)
            transpose_out(b - 1, 1 - slot)

    def body(k, _):
        half(k, 2 * k, 0)
        half(k, 2 * k + 1, 1)
        return 0

    lax.fori_loop(0, NPAIR, body, 0)

    wait_rows(1)
    transpose_out(NBLK - 1, 1)
    wait_hash(0)
    t_dma(0, 0, 0).wait()
    t_dma(1, 0, 0).wait()


@jax.jit
def _run(x_flat, weight, hash_keys):
    mesh = plsc.VectorSubcoreMesh(core_axis_name="c", subcore_axis_name="s")
    out = pl.kernel(
        _sc_kernel,
        out_type=jax.ShapeDtypeStruct((COLS, NCG, NRB, 8, RB), jnp.float32),
        mesh=mesh,
        compiler_params=pltpu.CompilerParams(
            use_tc_tiling_on_sc=False, needs_layout_passes=False),
        scratch_types=[
            pltpu.VMEM((XCHUNK,), jnp.int32),
            pltpu.VMEM((GSLOT,), jnp.int32),
            pltpu.VMEM((GSLOT,), jnp.int32),
            pltpu.VMEM((GSLOT,), jnp.int32),
            pltpu.VMEM((GSLOT,), jnp.int32),
            pltpu.VMEM((GSLOT,), jnp.int32),
            pltpu.VMEM((GSLOT,), jnp.int32),
            pltpu.VMEM((GSLOT, DIM), jnp.float32),
            pltpu.VMEM((GSLOT, DIM), jnp.float32),
            pltpu.VMEM((NCG, 8, RB), jnp.float32),
            pltpu.VMEM((NCG, 8, RB), jnp.float32),
            pltpu.SemaphoreType.DMA,
            pltpu.SemaphoreType.DMA,
            pltpu.SemaphoreType.DMA,
            pltpu.SemaphoreType.DMA,
            pltpu.SemaphoreType.DMA,
        ],
    )(x_flat, weight, hash_keys)
    return out


def kernel(x, weight, hash_keys):
    out5d = _run(x.reshape(ROWS * COLS), weight, hash_keys)
    return out5d.transpose(2, 4, 0, 1, 3).reshape(ROWS, COLS, DIM)


# speedup vs baseline: 1.1978x; 1.1978x over previous
import jax
import jax.numpy as jnp
from jax import lax
from jax.experimental import pallas as pl
from jax.experimental.pallas import tpu as pltpu
from jax.experimental.pallas import tpu_sc as plsc

VOCAB_SIZE = 1000000
DIM = 32
HASH_SIZE = int(VOCAB_SIZE * (1 - 0.95))

NC = 2
NS = 16
LANES = 16
NW = NC * NS

ROWS = 16384
COLS = 26
RB = 128
NRB = ROWS // RB
RB_PER_W = NRB // NW
XCHUNK = RB * COLS
NCG = 4
GSLOT = 8 * RB
NBLK = RB_PER_W * NCG
NPAIR = NBLK // 2


def _sc_kernel(x_hbm, w_hbm, h_hbm, o_hbm,
               xe, xg0, xg1, hg0, hg1, wg0, wg1, rows0, rows1, t0, t1,
               sem_h, sem_r0, sem_r1, sem_t0, sem_t1):
    wid = lax.axis_index("s") * NC + lax.axis_index("c")
    rb0 = wid * RB_PER_W
    iota = lax.iota(jnp.int32, LANES)
    iota26 = iota * 26
    div = jnp.full((LANES,), HASH_SIZE, jnp.int32)

    xg = (xg0, xg1)
    hg = (hg0, hg1)
    wg = (wg0, wg1)
    rows = (rows0, rows1)
    tb = (t0, t1)
    sem_r = (sem_r0, sem_r1)
    sem_t = (sem_t0, sem_t1)

    def stage_x(b):
        rb = jnp.minimum(rb0 + (b >> 2), NRB - 1)
        pltpu.sync_copy(x_hbm.at[pl.ds(rb * XCHUNK, XCHUNK)], xe)

    def extract(b, slot):
        cg = b & 3
        nv = jnp.minimum(COLS - cg * 8, 8)

        def valid(j, _):
            base = cg * 8 + j
            for r8 in range(8):
                idx = iota26 + (r8 * 16 * 26) + base
                v = plsc.load_gather(xe, [idx])
                xg[slot][pl.ds(j * RB + r8 * 16, LANES)] = v
            return 0

        def filler(j, _):
            sbase = (rb0 + (b >> 2)) * (NCG * GSLOT) + (b & 3) * GSLOT
            for r8 in range(8):
                xg[slot][pl.ds(j * RB + r8 * 16, LANES)] = (
                    iota + (sbase + j * RB + r8 * 16))
            return 0

        lax.fori_loop(0, nv, valid, 0)
        lax.fori_loop(nv, 8, filler, 0)

    def fire_hash(slot):
        def f(j, _):
            pltpu.async_copy(
                h_hbm.at[xg[slot].at[pl.ds(j * RB, RB)]],
                hg[slot].at[pl.ds(j * RB, RB)], sem_h)
            return 0
        lax.fori_loop(0, 8, f, 0)

    def wait_hash(slot):
        pltpu.make_async_copy(
            x_hbm.at[pl.ds(0, GSLOT)], hg[slot], sem_h).wait()

    def rem(slot):
        def f(i, _):
            v = hg[slot][pl.ds(i * LANES, LANES)]
            wg[slot][pl.ds(i * LANES, LANES)] = lax.rem(v, div)
            return 0
        lax.fori_loop(0, GSLOT // LANES, f, 0)

    def fire_rows(slot):
        def f(j, _):
            pltpu.async_copy(
                w_hbm.at[wg[slot].at[pl.ds(j * RB, RB)]],
                rows[slot].at[pl.ds(j * RB, RB)], sem_r[slot])
            return 0
        lax.fori_loop(0, 8, f, 0)

    def wait_rows(slot):
        pltpu.make_async_copy(
            w_hbm.at[pl.ds(0, GSLOT)], rows[slot], sem_r[slot]).wait()

    def t_dma(tslot, c, rb):
        return pltpu.make_async_copy(
            tb[tslot], o_hbm.at[c].at[:, rb], sem_t[tslot])

    i0_lo = lax.shift_right_logical(iota, 3)
    i0_hi = lax.shift_right_logical(iota + 16, 3)
    i1 = lax.bitwise_and(iota, 7)

    def transpose_out(b, slot):
        cg = b & 3
        rb = rb0 + (b >> 2)
        nv = jnp.minimum(COLS - cg * 8, 8)
        for j in range(8):
            @pl.when(j < nv)
            def _():
                t_dma(j % 2, 0, 0).wait()

                def srow(i, _):
                    for k in range(8):
                        rl = i * 8 + k
                        s = j * RB + rl
                        rlv = jnp.full((LANES,), rl, jnp.int32)
                        v0 = rows[slot][s, pl.ds(0, LANES)]
                        v1 = rows[slot][s, pl.ds(LANES, LANES)]
                        plsc.store_scatter(tb[j % 2], [i0_lo, i1, rlv], v0)
                        plsc.store_scatter(tb[j % 2], [i0_hi, i1, rlv], v1)
                    return 0

                lax.fori_loop(0, 16, srow, 0)
                t_dma(j % 2, cg * 8 + j, rb).start()

    t_dma(0, 0, rb0).start()
    t_dma(1, 1, rb0).start()
    stage_x(0)
    extract(0, 0)
    fire_hash(0)

    def half(k, b, slot):
        wait_hash(slot)

        @pl.when((b & 3) == 3)
        def _():
            stage_x(b + 1)
        extract(b + 1, 1 - slot)
        fire_hash(1 - slot)
        rem(slot)
        fire_rows(slot)

        @pl.when(b > 0)
        def _():
            wait_rows(1 - slot)
            transpose_out(b - 1, 1 - slot)

    def body(k, _):
        half(k, 2 * k, 0)
        half(k, 2 * k + 1, 1)
        return 0

    lax.fori_loop(0, NPAIR, body, 0)

    wait_rows(1)
    transpose_out(NBLK - 1, 1)
    wait_hash(0)
    t_dma(0, 0, 0).wait()
    t_dma(1, 0, 0).wait()


@jax.jit
def _run(x_flat, weight, hash_keys):
    mesh = plsc.VectorSubcoreMesh(core_axis_name="c", subcore_axis_name="s")
    out = pl.kernel(
        _sc_kernel,
        out_type=jax.ShapeDtypeStruct((COLS, NCG, NRB, 8, RB), jnp.float32),
        mesh=mesh,
        compiler_params=pltpu.CompilerParams(
            use_tc_tiling_on_sc=False, needs_layout_passes=False),
        scratch_types=[
            pltpu.VMEM((XCHUNK,), jnp.int32),
            pltpu.VMEM((GSLOT,), jnp.int32),
            pltpu.VMEM((GSLOT,), jnp.int32),
            pltpu.VMEM((GSLOT,), jnp.int32),
            pltpu.VMEM((GSLOT,), jnp.int32),
            pltpu.VMEM((GSLOT,), jnp.int32),
            pltpu.VMEM((GSLOT,), jnp.int32),
            pltpu.VMEM((GSLOT, DIM), jnp.float32),
            pltpu.VMEM((GSLOT, DIM), jnp.float32),
            pltpu.VMEM((NCG, 8, RB), jnp.float32),
            pltpu.VMEM((NCG, 8, RB), jnp.float32),
            pltpu.SemaphoreType.DMA,
            pltpu.SemaphoreType.DMA,
            pltpu.SemaphoreType.DMA,
            pltpu.SemaphoreType.DMA,
            pltpu.SemaphoreType.DMA,
        ],
    )(x_flat, weight, hash_keys)
    return out


def kernel(x, weight, hash_keys):
    out5d = _run(x.reshape(ROWS * COLS), weight, hash_keys)
    return out5d.transpose(2, 4, 0, 1, 3).reshape(ROWS, COLS, DIM)
